# baseline (speedup 1.0000x reference)
import functools

import jax
import jax.numpy as jnp
from jax import lax
from jax.experimental import pallas as pl
from jax.experimental.pallas import tpu as pltpu
from jax.experimental.pallas import tpu_sc as plsc

QUEUE = 65536
DIM = 128
BATCH = 4096
BS = 16384
NB = QUEUE // BS
WPB = BS // BATCH

LROWS = QUEUE // 128
LW = BATCH // 128
NWORK = 32
LPW = LROWS // NWORK


def _embed_kernel(s_ref, qe_ref, emb_ref, oe_ref):
    k = pl.program_id(0)
    s = s_ref[0]
    blk = s // WPB
    sub = jax.lax.rem(s, WPB)

    oe_ref[...] = qe_ref[...]

    @pl.when(k == blk)
    def _():
        x = emb_ref[...]
        n = jnp.sqrt(jnp.sum(x * x, axis=1, keepdims=True))
        oe_ref[pl.ds(sub * BATCH, BATCH), :] = x / jnp.maximum(n, 1e-12)


def _label_kernel(src_hbm, idx_hbm, out_hbm, idx_v, rows_v, sem):
    wid = lax.axis_index("s") * 2 + lax.axis_index("c")
    base = wid * LPW
    pltpu.sync_copy(idx_hbm.at[pl.ds(base, LPW)], idx_v)
    pltpu.async_copy(src_hbm.at[idx_v], rows_v, sem).wait()
    pltpu.sync_copy(rows_v, out_hbm.at[pl.ds(base, LPW), :])


def kernel(embeddings, labels, queue_embeds, queue_labels, queue_ptr):
    ldtype = queue_labels.dtype
    ptr = jax.lax.rem(queue_ptr.astype(jnp.int32), QUEUE)
    s_blk = jnp.reshape(ptr // BATCH, (1,))

    grid_spec = pltpu.PrefetchScalarGridSpec(
        num_scalar_prefetch=1,
        grid=(NB,),
        in_specs=[
            pl.BlockSpec((BS, DIM), lambda k, s: (k, 0)),
            pl.BlockSpec((BATCH, DIM), lambda k, s: (0, 0)),
        ],
        out_specs=pl.BlockSpec((BS, DIM), lambda k, s: (k, 0)),
    )
    new_qe = pl.pallas_call(
        _embed_kernel,
        grid_spec=grid_spec,
        out_shape=jax.ShapeDtypeStruct((QUEUE, DIM), queue_embeds.dtype),
    )(s_blk, queue_embeds, embeddings)

    ql2 = queue_labels.reshape(LROWS, 128)
    lab2 = labels.astype(ldtype).reshape(LW, 128)
    src = jnp.concatenate([lab2, ql2], axis=0)
    wbase = ptr // 128
    rows = jnp.arange(LROWS, dtype=jnp.int32)
    in_win = (rows >= wbase) & (rows < wbase + LW)
    idx = jnp.where(in_win, rows - wbase, rows + LW).astype(jnp.int32)

    lab_call = functools.partial(
        pl.kernel,
        out_type=jax.ShapeDtypeStruct((LROWS, 128), ldtype),
        mesh=plsc.VectorSubcoreMesh(core_axis_name="c", subcore_axis_name="s"),
        scratch_types=[
            pltpu.VMEM((LPW,), jnp.int32),
            pltpu.VMEM((LPW, 128), ldtype),
            pltpu.SemaphoreType.DMA,
        ],
    )
    new_ql2 = lab_call(_label_kernel)(src, idx)

    new_ptr = ((queue_ptr + BATCH) % QUEUE).astype(queue_ptr.dtype)
    return (new_qe, new_ql2.reshape(QUEUE), new_ptr)


# speedup vs baseline: 1.6067x; 1.6067x over previous
import jax
import jax.numpy as jnp
from jax.experimental import pallas as pl
from jax.experimental.pallas import tpu as pltpu

QUEUE = 65536
DIM = 128
BATCH = 4096
CH = 2048
NCHK = QUEUE // CH
NBUF = 6
RA = 4
LROWS = QUEUE // 128
LW = BATCH // 128


def _ring_kernel(s_ref, qe_ref, ql_ref, emb_ref, lab_ref, oe_ref, ol_ref,
                 bufs, lbuf, lwin, in_sems, out_sems, lsems):
    s4 = s_ref[0]
    wc = 2 * s4

    def in_q(i, b):
        return pltpu.make_async_copy(
            qe_ref.at[pl.ds(i * CH, CH), :], bufs.at[b], in_sems.at[b])

    def in_e(i, b):
        return pltpu.make_async_copy(
            emb_ref.at[pl.ds((i % 2) * CH, CH), :], bufs.at[b], in_sems.at[b])

    def out_c(i, b):
        return pltpu.make_async_copy(
            bufs.at[b], oe_ref.at[pl.ds(i * CH, CH), :], out_sems.at[b])

    def start_in(i, b):
        is_win = (i == wc) | (i == wc + 1)

        @pl.when(is_win)
        def _():
            in_e(i, b).start()

        @pl.when(jnp.logical_not(is_win))
        def _():
            in_q(i, b).start()

    lin1 = pltpu.make_async_copy(ql_ref, lbuf, lsems.at[0])
    lin2 = pltpu.make_async_copy(lab_ref, lwin, lsems.at[1])
    lout = pltpu.make_async_copy(lbuf, ol_ref, lsems.at[2])
    lin1.start()
    lin2.start()

    for i in range(RA):
        start_in(i, i % NBUF)

    lin1.wait()
    lin2.wait()
    lbuf[pl.ds(s4 * LW, LW), :] = lwin[...]
    lout.start()

    for i in range(NCHK):
        b = i % NBUF
        in_q(i, b).wait()

        is_win = (i == wc) | (i == wc + 1)

        @pl.when(is_win)
        def _(b=b):
            x = bufs[b]
            n = jnp.sqrt(jnp.sum(x * x, axis=1, keepdims=True))
            bufs[b] = x / jnp.maximum(n, 1e-12)

        out_c(i, b).start()
        j = i + RA
        if j < NCHK:
            bj = j % NBUF
            if j - NBUF >= 0:
                out_c(j - NBUF, bj).wait()
            start_in(j, bj)

    for i in range(NCHK - NBUF, NCHK):
        out_c(i, i % NBUF).wait()
    lout.wait()


def kernel(embeddings, labels, queue_embeds, queue_labels, queue_ptr):
    ldtype = queue_labels.dtype
    ql2 = queue_labels.reshape(LROWS, 128)
    lab2 = labels.astype(ldtype).reshape(LW, 128)
    s_blk = jnp.reshape(
        jax.lax.rem(queue_ptr.astype(jnp.int32) // BATCH, QUEUE // BATCH), (1,)
    )

    grid_spec = pltpu.PrefetchScalarGridSpec(
        num_scalar_prefetch=1,
        grid=(1,),
        in_specs=[
            pl.BlockSpec(memory_space=pl.ANY),
            pl.BlockSpec(memory_space=pl.ANY),
            pl.BlockSpec(memory_space=pl.ANY),
            pl.BlockSpec(memory_space=pl.ANY),
        ],
        out_specs=[
            pl.BlockSpec(memory_space=pl.ANY),
            pl.BlockSpec(memory_space=pl.ANY),
        ],
        scratch_shapes=[
            pltpu.VMEM((NBUF, CH, DIM), jnp.float32),
            pltpu.VMEM((LROWS, 128), ldtype),
            pltpu.VMEM((LW, 128), ldtype),
            pltpu.SemaphoreType.DMA((NBUF,)),
            pltpu.SemaphoreType.DMA((NBUF,)),
            pltpu.SemaphoreType.DMA((3,)),
        ],
    )

    new_qe, new_ql2 = pl.pallas_call(
        _ring_kernel,
        grid_spec=grid_spec,
        out_shape=[
            jax.ShapeDtypeStruct((QUEUE, DIM), queue_embeds.dtype),
            jax.ShapeDtypeStruct((LROWS, 128), ldtype),
        ],
    )(s_blk, queue_embeds, ql2, embeddings, lab2)

    new_ptr = ((queue_ptr + BATCH) % QUEUE).astype(queue_ptr.dtype)
    return (new_qe, new_ql2.reshape(QUEUE), new_ptr)


# speedup vs baseline: 1.6787x; 1.0448x over previous
import jax
import jax.numpy as jnp
from jax.experimental import pallas as pl
from jax.experimental.pallas import tpu as pltpu

QUEUE = 65536
DIM = 128
BATCH = 4096
CH = 4096
NCHK = QUEUE // CH
NBUF = 6
RA = 4
LROWS = QUEUE // 128
LW = BATCH // 128


def _ring_kernel(s_ref, qe_ref, ql_ref, emb_ref, lab_ref, oe_ref, ol_ref,
                 bufs, lbuf, lwin, in_sems, out_sems, lsems):
    s4 = s_ref[0]
    wc = s4

    def in_q(i, b):
        return pltpu.make_async_copy(
            qe_ref.at[pl.ds(i * CH, CH), :], bufs.at[b], in_sems.at[b])

    def in_e(i, b):
        return pltpu.make_async_copy(
            emb_ref.at[pl.ds(0, CH), :], bufs.at[b], in_sems.at[b])

    def out_c(i, b):
        return pltpu.make_async_copy(
            bufs.at[b], oe_ref.at[pl.ds(i * CH, CH), :], out_sems.at[b])

    def start_in(i, b):
        is_win = i == wc

        @pl.when(is_win)
        def _():
            in_e(i, b).start()

        @pl.when(jnp.logical_not(is_win))
        def _():
            in_q(i, b).start()

    lin1 = pltpu.make_async_copy(ql_ref, lbuf, lsems.at[0])
    lin2 = pltpu.make_async_copy(lab_ref, lwin, lsems.at[1])
    lout = pltpu.make_async_copy(lbuf, ol_ref, lsems.at[2])
    lin1.start()
    lin2.start()

    for i in range(RA):
        start_in(i, i % NBUF)

    lin1.wait()
    lin2.wait()
    lbuf[pl.ds(s4 * LW, LW), :] = lwin[...]
    lout.start()

    for i in range(NCHK):
        b = i % NBUF
        in_q(i, b).wait()

        is_win = i == wc

        @pl.when(is_win)
        def _(b=b):
            x = bufs[b]
            n = jnp.sqrt(jnp.sum(x * x, axis=1, keepdims=True))
            bufs[b] = x / jnp.maximum(n, 1e-12)

        out_c(i, b).start()
        j = i + RA
        if j < NCHK:
            bj = j % NBUF
            if j - NBUF >= 0:
                out_c(j - NBUF, bj).wait()
            start_in(j, bj)

    for i in range(NCHK - NBUF, NCHK):
        out_c(i, i % NBUF).wait()
    lout.wait()


def kernel(embeddings, labels, queue_embeds, queue_labels, queue_ptr):
    ldtype = queue_labels.dtype
    ql2 = queue_labels.reshape(LROWS, 128)
    lab2 = labels.astype(ldtype).reshape(LW, 128)
    s_blk = jnp.reshape(
        jax.lax.rem(queue_ptr.astype(jnp.int32) // BATCH, QUEUE // BATCH), (1,)
    )

    grid_spec = pltpu.PrefetchScalarGridSpec(
        num_scalar_prefetch=1,
        grid=(1,),
        in_specs=[
            pl.BlockSpec(memory_space=pl.ANY),
            pl.BlockSpec(memory_space=pl.ANY),
            pl.BlockSpec(memory_space=pl.ANY),
            pl.BlockSpec(memory_space=pl.ANY),
        ],
        out_specs=[
            pl.BlockSpec(memory_space=pl.ANY),
            pl.BlockSpec(memory_space=pl.ANY),
        ],
        scratch_shapes=[
            pltpu.VMEM((NBUF, CH, DIM), jnp.float32),
            pltpu.VMEM((LROWS, 128), ldtype),
            pltpu.VMEM((LW, 128), ldtype),
            pltpu.SemaphoreType.DMA((NBUF,)),
            pltpu.SemaphoreType.DMA((NBUF,)),
            pltpu.SemaphoreType.DMA((3,)),
        ],
    )

    new_qe, new_ql2 = pl.pallas_call(
        _ring_kernel,
        grid_spec=grid_spec,
        out_shape=[
            jax.ShapeDtypeStruct((QUEUE, DIM), queue_embeds.dtype),
            jax.ShapeDtypeStruct((LROWS, 128), ldtype),
        ],
    )(s_blk, queue_embeds, ql2, embeddings, lab2)

    new_ptr = ((queue_ptr + BATCH) % QUEUE).astype(queue_ptr.dtype)
    return (new_qe, new_ql2.reshape(QUEUE), new_ptr)


# speedup vs baseline: 1.7188x; 1.0239x over previous
import jax
import jax.numpy as jnp
from jax.experimental import pallas as pl
from jax.experimental.pallas import tpu as pltpu

QUEUE = 65536
DIM = 128
BATCH = 4096
BS = 16384
NB = QUEUE // BS
WPB = BS // BATCH
LBS = BS // 128
LW = BATCH // 128


def _enqueue_kernel(s_ref, qe_ref, ql_ref, emb_ref, lab_ref, oe_ref, ol_ref):
    k = pl.program_id(0)
    s = s_ref[0]
    blk = s // WPB
    sub = jax.lax.rem(s, WPB)

    oe_ref[...] = qe_ref[...]
    ol_ref[...] = ql_ref[...]

    @pl.when(k == blk)
    def _():
        x = emb_ref[...]
        n = jnp.sqrt(jnp.sum(x * x, axis=1, keepdims=True))
        oe_ref[pl.ds(sub * BATCH, BATCH), :] = x / jnp.maximum(n, 1e-12)
        ol_ref[pl.ds(sub * LW, LW), :] = lab_ref[...]


def kernel(embeddings, labels, queue_embeds, queue_labels, queue_ptr):
    ldtype = queue_labels.dtype
    ql2 = queue_labels.reshape(QUEUE // 128, 128)
    lab2 = labels.astype(ldtype).reshape(LW, 128)
    s_blk = jnp.reshape(
        jax.lax.rem(queue_ptr.astype(jnp.int32) // BATCH, QUEUE // BATCH), (1,)
    )

    grid_spec = pltpu.PrefetchScalarGridSpec(
        num_scalar_prefetch=1,
        grid=(NB,),
        in_specs=[
            pl.BlockSpec((BS, DIM), lambda k, s: (k, 0)),
            pl.BlockSpec((LBS, 128), lambda k, s: (k, 0)),
            pl.BlockSpec((BATCH, DIM), lambda k, s: (0, 0)),
            pl.BlockSpec((LW, 128), lambda k, s: (0, 0)),
        ],
        out_specs=[
            pl.BlockSpec((BS, DIM), lambda k, s: (k, 0)),
            pl.BlockSpec((LBS, 128), lambda k, s: (k, 0)),
        ],
    )

    new_qe, new_ql2 = pl.pallas_call(
        _enqueue_kernel,
        grid_spec=grid_spec,
        out_shape=[
            jax.ShapeDtypeStruct((QUEUE, DIM), queue_embeds.dtype),
            jax.ShapeDtypeStruct((QUEUE // 128, 128), ldtype),
        ],
    )(s_blk, queue_embeds, ql2, embeddings, lab2)

    new_ptr = ((queue_ptr + BATCH) % QUEUE).astype(queue_ptr.dtype)
    return (new_qe, new_ql2.reshape(QUEUE), new_ptr)
